# Initial kernel scaffold; baseline (speedup 1.0000x reference)
import jax
import jax.numpy as jnp
from jax.experimental import pallas as pl


def kernel(input_ids, token_type_ids, W_word, W_pos, W_type, gamma, beta):
    raise NotImplementedError("write your pallas kernel here")


# speedup vs baseline: 3.9363x; 3.9363x over previous
import functools

import jax
import jax.numpy as jnp
from jax import lax
from jax.experimental import pallas as pl
from jax.experimental.pallas import tpu as pltpu
from jax.experimental.pallas import tpu_sc as plsc

VOCAB = 100000
HIDDEN = 128
SEQ = 200
EPS = 1e-12
LANES = 16
NSL = HIDDEN // LANES
NC, NS = 2, 16
NW = NC * NS
NSEQ = 1024
SEQ_PER_W = NSEQ // NW
CH = 40
NCH = SEQ // CH


def _hsum(x):
    lanes = lax.iota(jnp.int32, LANES)
    for sh in (8, 4, 2, 1):
        idx = jnp.bitwise_xor(lanes, sh)
        x = x + x.at[idx].get(mode="promise_in_bounds")
    return x


def _rsqrt(v):
    i = lax.bitcast_convert_type(v, jnp.int32)
    i = jnp.int32(0x5F3759DF) - lax.shift_right_arithmetic(i, 1)
    y = lax.bitcast_convert_type(i, jnp.float32)
    for _ in range(3):
        y = y * (1.5 - 0.5 * v * y * y)
    return y


@functools.partial(
    pl.kernel,
    out_type=jax.ShapeDtypeStruct((NSEQ * SEQ, HIDDEN), jnp.float32),
    mesh=plsc.VectorSubcoreMesh(
        core_axis_name="c", subcore_axis_name="s", num_cores=NC, num_subcores=NS
    ),
    scratch_types=[
        pltpu.VMEM((SEQ, HIDDEN), jnp.float32),
        pltpu.VMEM((2, HIDDEN), jnp.float32),
        pltpu.VMEM((HIDDEN,), jnp.float32),
        pltpu.VMEM((HIDDEN,), jnp.float32),
        pltpu.VMEM((NCH, CH), jnp.int32),
        pltpu.VMEM((SEQ + LANES,), jnp.int32),
        pltpu.VMEM((SEQ, HIDDEN), jnp.float32),
        pltpu.SemaphoreType.DMA,
    ],
)
def _emb_kernel(ids_hbm, tt_hbm, wword_hbm, wpos_hbm, wtype_hbm, gamma_hbm,
                beta_hbm, out_hbm, pos_v, type_v, gamma_v, beta_v, idx2, tt_v,
                rows_v, sem):
    wid = lax.axis_index("s") * NC + lax.axis_index("c")

    pltpu.sync_copy(wpos_hbm.at[pl.ds(0, SEQ)], pos_v)
    pltpu.sync_copy(wtype_hbm, type_v)
    pltpu.sync_copy(gamma_hbm, gamma_v)
    pltpu.sync_copy(beta_hbm, beta_v)

    t0 = [type_v[0, pl.ds(j * LANES, LANES)] for j in range(NSL)]
    td = [type_v[1, pl.ds(j * LANES, LANES)] - t0[j] for j in range(NSL)]
    gam = [gamma_v[pl.ds(j * LANES, LANES)] for j in range(NSL)]
    bet = [beta_v[pl.ds(j * LANES, LANES)] for j in range(NSL)]

    def seq_body(s_i, carry):
        tokbase = (wid * SEQ_PER_W + s_i) * SEQ
        pltpu.sync_copy(tt_hbm.at[pl.ds(tokbase, SEQ)], tt_v.at[pl.ds(0, SEQ)])
        for c in range(NCH):
            pltpu.sync_copy(ids_hbm.at[pl.ds(tokbase + c * CH, CH)], idx2.at[c])
        copies = [
            pltpu.async_copy(
                wword_hbm.at[idx2.at[c]], rows_v.at[pl.ds(c * CH, CH)], sem
            )
            for c in range(NCH)
        ]
        for cp in copies:
            cp.wait()

        def tok_body(i, tcarry):
            ttf = tt_v[pl.ds(i, LANES)][0].astype(jnp.float32)
            x = []
            sv = None
            qv = None
            for j in range(NSL):
                sl = pl.ds(j * LANES, LANES)
                xj = rows_v[i, sl] + pos_v[i, sl] + t0[j] + ttf * td[j]
                x.append(xj)
                sv = xj if sv is None else sv + xj
                qv = xj * xj if qv is None else qv + xj * xj
            mean = _hsum(sv) * (1.0 / HIDDEN)
            var = _hsum(qv) * (1.0 / HIDDEN) - mean * mean
            rstd = _rsqrt(var + EPS)
            for j in range(NSL):
                sl = pl.ds(j * LANES, LANES)
                rows_v[i, sl] = (x[j] - mean) * rstd * gam[j] + bet[j]
            return tcarry

        lax.fori_loop(0, SEQ, tok_body, 0)
        pltpu.sync_copy(rows_v, out_hbm.at[pl.ds(tokbase, SEQ)])
        return carry

    lax.fori_loop(0, SEQ_PER_W, seq_body, 0)


def kernel(input_ids, token_type_ids, W_word, W_pos, W_type, gamma, beta):
    b, s = input_ids.shape
    ids = input_ids.reshape(-1).astype(jnp.int32)
    tt = token_type_ids.reshape(-1).astype(jnp.int32)
    out = _emb_kernel(ids, tt, W_word, W_pos, W_type, gamma, beta)
    return out.reshape(b, s, HIDDEN)


# speedup vs baseline: 6.3480x; 1.6127x over previous
import functools

import jax
import jax.numpy as jnp
from jax import lax
from jax.experimental import pallas as pl
from jax.experimental.pallas import tpu as pltpu
from jax.experimental.pallas import tpu_sc as plsc

VOCAB = 100000
HIDDEN = 128
SEQ = 200
EPS = 1e-12
LANES = 16
NSL = HIDDEN // LANES
NC, NS = 2, 16
NW = NC * NS
NSEQ = 1024
SEQ_PER_W = NSEQ // NW
NPAIR = SEQ_PER_W // 2
CH = 40
NCH = SEQ // CH
SGRP = (SEQ + LANES - 1) // LANES
SEQP = SGRP * LANES
MID = SGRP // 2
TTROW = 256


def _hsum(x):
    lanes = lax.iota(jnp.int32, LANES)
    for sh in (8, 4, 2, 1):
        idx = jnp.bitwise_xor(lanes, sh)
        x = x + x.at[idx].get(mode="promise_in_bounds")
    return x


def _rsqrt(v):
    i = lax.bitcast_convert_type(v, jnp.int32)
    i = jnp.int32(0x5F3759DF) - lax.shift_right_arithmetic(i, 1)
    y = lax.bitcast_convert_type(i, jnp.float32)
    for _ in range(2):
        y = y * (1.5 - 0.5 * v * y * y)
    return y


@functools.partial(
    pl.kernel,
    out_type=jax.ShapeDtypeStruct((NSEQ * SEQ, HIDDEN), jnp.float32),
    mesh=plsc.VectorSubcoreMesh(
        core_axis_name="c", subcore_axis_name="s", num_cores=NC, num_subcores=NS
    ),
    scratch_types=[
        pltpu.VMEM((SEQP, HIDDEN), jnp.float32),
        pltpu.VMEM((2, HIDDEN), jnp.float32),
        pltpu.VMEM((HIDDEN,), jnp.float32),
        pltpu.VMEM((HIDDEN,), jnp.float32),
        pltpu.VMEM((NCH, CH), jnp.int32),
        pltpu.VMEM((NCH, CH), jnp.int32),
        pltpu.VMEM((TTROW,), jnp.int32),
        pltpu.VMEM((TTROW,), jnp.int32),
        pltpu.VMEM((2, SEQP, HIDDEN), jnp.float32),
        pltpu.SemaphoreType.DMA,
        pltpu.SemaphoreType.DMA,
        pltpu.SemaphoreType.DMA,
        pltpu.SemaphoreType.DMA,
        pltpu.SemaphoreType.DMA,
        pltpu.SemaphoreType.DMA,
        pltpu.SemaphoreType.DMA,
        pltpu.SemaphoreType.DMA,
    ],
)
def _emb_kernel(ids_hbm, tt_hbm, wword_hbm, wpos_hbm, wtype_hbm, gamma_hbm,
                beta_hbm, out_hbm, pos_v, type_v, gamma_v, beta_v, idx0_v,
                idx1_v, tt0_v, tt1_v, rows_v, sem_g0, sem_g1, sem_i0, sem_i1,
                sem_t0, sem_t1, sem_o0, sem_o1):
    wid = lax.axis_index("s") * NC + lax.axis_index("c")
    wbase = wid * SEQ_PER_W * SEQ

    sem_g = (sem_g0, sem_g1)
    sem_i = (sem_i0, sem_i1)
    sem_t = (sem_t0, sem_t1)
    sem_o = (sem_o0, sem_o1)
    rows = (rows_v.at[0], rows_v.at[1])
    idxb = (idx0_v, idx1_v)
    ttb = (tt0_v, tt1_v)

    def issue_gather(b):
        for c in range(NCH):
            pltpu.async_copy(
                wword_hbm.at[idxb[b].at[c]],
                rows[b].at[pl.ds(c * CH, CH)],
                sem_g[b],
            )

    def wait_gather(b):
        pltpu.make_async_copy(
            out_hbm.at[pl.ds(0, SEQ)], rows[b].at[pl.ds(0, SEQ)], sem_g[b]
        ).wait()

    def issue_idx(b, tokbase):
        for c in range(NCH):
            pltpu.async_copy(
                ids_hbm.at[pl.ds(tokbase + c * CH, CH)], idxb[b].at[c], sem_i[b]
            )

    def wait_idx(b):
        for c in range(NCH):
            pltpu.make_async_copy(
                ids_hbm.at[pl.ds(0, CH)], idxb[b].at[c], sem_i[b]
            ).wait()

    def issue_tt(b, tokbase):
        pltpu.async_copy(tt_hbm.at[pl.ds(tokbase, TTROW)], ttb[b], sem_t[b])

    def wait_tt(b):
        pltpu.make_async_copy(
            tt_hbm.at[pl.ds(0, TTROW)], ttb[b], sem_t[b]
        ).wait()

    def issue_out(b, tokbase):
        pltpu.async_copy(
            rows[b].at[pl.ds(0, SEQ)], out_hbm.at[pl.ds(tokbase, SEQ)], sem_o[b]
        )

    def wait_out(b):
        pltpu.make_async_copy(
            rows[b].at[pl.ds(0, SEQ)], out_hbm.at[pl.ds(0, SEQ)], sem_o[b]
        ).wait()

    for c in range(NCH):
        pltpu.sync_copy(ids_hbm.at[pl.ds(wbase + c * CH, CH)], idxb[0].at[c])
    issue_gather(0)
    issue_tt(0, wbase)
    issue_idx(1, wbase + SEQ)

    pltpu.sync_copy(wpos_hbm.at[pl.ds(0, SEQ)], pos_v.at[pl.ds(0, SEQ)])
    pltpu.sync_copy(wtype_hbm, type_v)
    pltpu.sync_copy(gamma_hbm, gamma_v)
    pltpu.sync_copy(beta_hbm, beta_v)

    t0 = [type_v[0, pl.ds(j * LANES, LANES)] for j in range(NSL)]
    td = [type_v[1, pl.ds(j * LANES, LANES)] - t0[j] for j in range(NSL)]
    gam = [gamma_v[pl.ds(j * LANES, LANES)] for j in range(NSL)]
    bet = [beta_v[pl.ds(j * LANES, LANES)] for j in range(NSL)]

    def pt_body(i, carry):
        for j in range(NSL):
            sl = pl.ds(j * LANES, LANES)
            pos_v[i, sl] = pos_v[i, sl] + t0[j]
        return carry

    lax.fori_loop(0, SEQ, pt_body, 0)

    def make_sg_body(b, mid_work):
        rb, tb = rows[b], ttb[b]

        def sg_body(sg, carry):
            @pl.when(sg == MID)
            def _():
                mid_work()

            base = pl.multiple_of(sg * LANES, LANES)
            tts = tb[pl.ds(base, LANES)]
            for k in range(LANES):
                i = base + k
                ttf = tts[k].astype(jnp.float32)
                x = []
                sv = None
                qv = None
                for j in range(NSL):
                    sl = pl.ds(j * LANES, LANES)
                    xj = rb[i, sl] + (pos_v[i, sl] + ttf * td[j])
                    x.append(xj)
                    sv = xj if sv is None else sv + xj
                    qv = xj * xj if qv is None else qv + xj * xj
                mean = _hsum(sv) * (1.0 / HIDDEN)
                var = _hsum(qv) * (1.0 / HIDDEN) - mean * mean
                rstd = _rsqrt(var + EPS)
                for j in range(NSL):
                    sl = pl.ds(j * LANES, LANES)
                    rb[i, sl] = (x[j] - mean) * (rstd * gam[j]) + bet[j]
            return carry

        return sg_body

    def pair_body(g, carry):
        pbase = wbase + 2 * g * SEQ

        def mid0():
            wait_idx(1)

            @pl.when(g > 0)
            def _():
                wait_out(1)

            issue_gather(1)

            @pl.when(g < NPAIR - 1)
            def _():
                issue_idx(0, pbase + 2 * SEQ)

            issue_tt(1, pbase + SEQ)

        wait_gather(0)
        wait_tt(0)
        lax.fori_loop(0, SGRP, make_sg_body(0, mid0), 0)
        issue_out(0, pbase)

        def mid1():
            wait_out(0)

            @pl.when(g < NPAIR - 1)
            def _():
                wait_idx(0)
                issue_gather(0)
                issue_idx(1, pbase + 3 * SEQ)
                issue_tt(0, pbase + 2 * SEQ)

        wait_gather(1)
        wait_tt(1)
        lax.fori_loop(0, SGRP, make_sg_body(1, mid1), 0)
        issue_out(1, pbase + SEQ)
        return carry

    lax.fori_loop(0, NPAIR, pair_body, 0)
    wait_out(1)


def kernel(input_ids, token_type_ids, W_word, W_pos, W_type, gamma, beta):
    b, s = input_ids.shape
    ids = input_ids.reshape(-1).astype(jnp.int32)
    tt = jnp.pad(token_type_ids.reshape(-1).astype(jnp.int32), (0, TTROW - SEQ))
    out = _emb_kernel(ids, tt, W_word, W_pos, W_type, gamma, beta)
    return out.reshape(b, s, HIDDEN)
